# Initial kernel scaffold; baseline (speedup 1.0000x reference)
import jax
import jax.numpy as jnp
from jax.experimental import pallas as pl


def kernel(ast_nodes_encodings, ast_node_idx_to_pdg_node_idx_mapping_key, ast_node_idx_to_pdg_node_idx_mapping_value, pdg_node_idx_to_sub_ast_root_idx_mapping_key, pdg_node_idx_to_sub_ast_root_idx_mapping_value, nr_cfg_nodes):
    raise NotImplementedError("write your pallas kernel here")


# speedup vs baseline: 2.7233x; 2.7233x over previous
import functools

import jax
import jax.numpy as jnp
from jax import lax
from jax.experimental import pallas as pl
from jax.experimental.pallas import tpu as pltpu
from jax.experimental.pallas import tpu_sc as plsc

_K = 128


def _build_sc_kernel(n_tab2, d2, e_pad, n_cfg, n_acc, n_chunks, rows_main,
                     rows_last, zrows):
  mesh = plsc.VectorSubcoreMesh(core_axis_name="c", subcore_axis_name="s")

  @functools.partial(
      pl.kernel,
      mesh=mesh,
      out_type=jax.ShapeDtypeStruct((2, n_cfg, d2), jnp.float32),
      scratch_types=[
          pltpu.VMEM((_K,), jnp.int32),
          pltpu.VMEM((_K,), jnp.int32),
          pltpu.VMEM((_K,), jnp.int32),
          pltpu.VMEM((_K, d2), jnp.float32),
          pltpu.VMEM_SHARED((n_acc, d2), jnp.float32),
          pltpu.SemaphoreType.DMA,
      ],
  )
  def body(table_hbm, keys_hbm, segs_hbm, zeros_hbm, out_hbm,
           keyv, gidxv, segv, rows, acc, gsem):
    c = lax.axis_index("c")
    s = lax.axis_index("s")

    pltpu.sync_copy(zeros_hbm, acc.at[pl.ds(s * zrows, zrows)])
    plsc.subcore_barrier()

    def step(j, carry):
      base = pl.multiple_of((j * 16 + s) * _K, _K)
      pltpu.sync_copy(keys_hbm.at[pl.ds(base, _K)], keyv)
      pltpu.sync_copy(segs_hbm.at[pl.ds(base, _K)], segv)
      for i in range(_K // 16):
        sl = pl.ds(i * 16, 16)
        gidxv[sl] = keyv[sl] * 2 + c
      pltpu.async_copy(table_hbm.at[gidxv], rows, gsem).wait()
      pltpu.sync_copy(rows, acc.at[segv], add=True)
      return carry

    lax.fori_loop(0, n_chunks, step, 0)
    plsc.subcore_barrier()

    @pl.when(s < 15)
    def _():
      r0 = pl.multiple_of(s * rows_main, 8)
      pltpu.sync_copy(acc.at[pl.ds(r0, rows_main)],
                      out_hbm.at[c, pl.ds(r0, rows_main)])

    @pl.when(s == 15)
    def _():
      r0 = 15 * rows_main
      pltpu.sync_copy(acc.at[pl.ds(r0, rows_last)],
                      out_hbm.at[c, pl.ds(r0, rows_last)])

  return body


def kernel(ast_nodes_encodings,
           ast_node_idx_to_pdg_node_idx_mapping_key,
           ast_node_idx_to_pdg_node_idx_mapping_value,
           pdg_node_idx_to_sub_ast_root_idx_mapping_key,
           pdg_node_idx_to_sub_ast_root_idx_mapping_value,
           nr_cfg_nodes):
  table = ast_nodes_encodings
  keys = ast_node_idx_to_pdg_node_idx_mapping_key
  segs = ast_node_idx_to_pdg_node_idx_mapping_value
  n_ast, d = table.shape
  d2 = d // 2
  e = keys.shape[0]
  n_cfg = pdg_node_idx_to_sub_ast_root_idx_mapping_key.shape[0]

  ch = 16 * _K
  n_chunks = -(-e // ch)
  e_pad = n_chunks * ch
  pad = e_pad - e
  keys_p = jnp.concatenate(
      [keys.astype(jnp.int32), jnp.zeros((pad,), jnp.int32)])
  segs_p = jnp.concatenate(
      [segs.astype(jnp.int32), jnp.full((pad,), n_cfg, jnp.int32)])
  table_flat = table.reshape(n_ast * 2, d2)

  zrows = -(-(n_cfg + 1) // (16 * 8)) * 8
  n_acc = 16 * zrows
  rows_main = (n_cfg // (16 * 8)) * 8
  rows_last = n_cfg - 15 * rows_main
  zeros = jnp.zeros((zrows, d2), jnp.float32)

  body = _build_sc_kernel(n_ast * 2, d2, e_pad, n_cfg, n_acc, n_chunks,
                          rows_main, rows_last, zrows)
  out = body(table_flat, keys_p, segs_p, zeros)
  return jnp.concatenate([out[0], out[1]], axis=-1)


# speedup vs baseline: 3.5016x; 1.2858x over previous
import functools

import jax
import jax.numpy as jnp
from jax import lax
from jax.experimental import pallas as pl
from jax.experimental.pallas import tpu as pltpu
from jax.experimental.pallas import tpu_sc as plsc

_K = 128


def _build_sc_kernel(d2, n_cfg, n_acc, n_pairs, rows_main, rows_last, zrows):
  mesh = plsc.VectorSubcoreMesh(core_axis_name="c", subcore_axis_name="s")

  @functools.partial(
      pl.kernel,
      mesh=mesh,
      out_type=jax.ShapeDtypeStruct((2, n_cfg, d2), jnp.float32),
      scratch_types=[
          pltpu.VMEM((2, _K), jnp.int32),
          pltpu.VMEM((2, _K), jnp.int32),
          pltpu.VMEM((_K,), jnp.int32),
          pltpu.VMEM((_K,), jnp.int32),
          pltpu.VMEM((_K, d2), jnp.float32),
          pltpu.VMEM((_K, d2), jnp.float32),
          pltpu.VMEM_SHARED((n_acc, d2), jnp.float32),
          pltpu.SemaphoreType.DMA,
          pltpu.SemaphoreType.DMA,
      ],
  )
  def body(table_hbm, ks_hbm, zeros_hbm, out_hbm,
           ks0, ks1, gidx0, gidx1, rows0, rows1, acc, sem0, sem1):
    c = lax.axis_index("c")
    s = lax.axis_index("s")

    pltpu.sync_copy(zeros_hbm, acc.at[pl.ds(s * zrows, zrows)])
    plsc.subcore_barrier()

    def prep(chunk, ksb, gidxb, rowsb, sem):
      pltpu.sync_copy(ks_hbm.at[chunk], ksb)
      for i in range(_K // 16):
        sl = pl.ds(i * 16, 16)
        gidxb[sl] = ksb[0, sl] * 2 + c
      return pltpu.async_copy(table_hbm.at[gidxb], rowsb, sem)

    prep(s, ks0, gidx0, rows0, sem0)

    def step(t, carry):
      a = (2 * t) * 16 + s
      b = (2 * t + 1) * 16 + s
      prep(b, ks1, gidx1, rows1, sem1)
      pltpu.make_async_copy(table_hbm.at[gidx0], rows0, sem0).wait()
      pltpu.sync_copy(rows0, acc.at[ks0.at[1]], add=True)

      @pl.when(t < n_pairs - 1)
      def _():
        prep(b + 16, ks0, gidx0, rows0, sem0)

      pltpu.make_async_copy(table_hbm.at[gidx1], rows1, sem1).wait()
      pltpu.sync_copy(rows1, acc.at[ks1.at[1]], add=True)
      return carry

    lax.fori_loop(0, n_pairs, step, 0)
    plsc.subcore_barrier()

    @pl.when(s < 15)
    def _():
      r0 = pl.multiple_of(s * rows_main, 8)
      pltpu.sync_copy(acc.at[pl.ds(r0, rows_main)],
                      out_hbm.at[c, pl.ds(r0, rows_main)])

    @pl.when(s == 15)
    def _():
      r0 = 15 * rows_main
      pltpu.sync_copy(acc.at[pl.ds(r0, rows_last)],
                      out_hbm.at[c, pl.ds(r0, rows_last)])

  return body


def kernel(ast_nodes_encodings,
           ast_node_idx_to_pdg_node_idx_mapping_key,
           ast_node_idx_to_pdg_node_idx_mapping_value,
           pdg_node_idx_to_sub_ast_root_idx_mapping_key,
           pdg_node_idx_to_sub_ast_root_idx_mapping_value,
           nr_cfg_nodes):
  table = ast_nodes_encodings
  keys = ast_node_idx_to_pdg_node_idx_mapping_key
  segs = ast_node_idx_to_pdg_node_idx_mapping_value
  n_ast, d = table.shape
  d2 = d // 2
  e = keys.shape[0]
  n_cfg = pdg_node_idx_to_sub_ast_root_idx_mapping_key.shape[0]

  ch = 16 * _K
  n_rounds = -(-e // (2 * ch)) * 2
  n_chunks = n_rounds * 16
  e_pad = n_rounds * ch
  pad = e_pad - e
  keys_p = jnp.concatenate(
      [keys.astype(jnp.int32), jnp.zeros((pad,), jnp.int32)])
  segs_p = jnp.concatenate(
      [segs.astype(jnp.int32), jnp.full((pad,), n_cfg, jnp.int32)])
  ks = jnp.stack(
      [keys_p.reshape(n_chunks, _K), segs_p.reshape(n_chunks, _K)], axis=1)
  table_flat = table.reshape(n_ast * 2, d2)

  zrows = -(-(n_cfg + 1) // (16 * 8)) * 8
  n_acc = 16 * zrows
  rows_main = (n_cfg // (16 * 8)) * 8
  rows_last = n_cfg - 15 * rows_main
  zeros = jnp.zeros((zrows, d2), jnp.float32)

  body = _build_sc_kernel(d2, n_cfg, n_acc, n_rounds // 2, rows_main,
                          rows_last, zrows)
  out = body(table_flat, ks, zeros)
  return jnp.concatenate([out[0], out[1]], axis=-1)
